# baseline (speedup 1.0000x reference)
import jax
import jax.numpy as jnp
from jax.experimental import pallas as pl

_TN = 512


def _logits_kernel(x_ref, m_ref, o_ref):
    o_ref[...] = jax.lax.dot_general(
        x_ref[...],
        m_ref[...],
        dimension_numbers=(((1,), (1,)), ((), ())),
        preferred_element_type=jnp.float32,
    )


def kernel(inputs, targets, mem, epoch):
    del targets, epoch
    m, k = inputs.shape
    n = mem.shape[0]
    return pl.pallas_call(
        _logits_kernel,
        grid=(pl.cdiv(n, _TN),),
        in_specs=[
            pl.BlockSpec((m, k), lambda i: (0, 0)),
            pl.BlockSpec((_TN, k), lambda i: (i, 0)),
        ],
        out_specs=pl.BlockSpec((m, _TN), lambda i: (0, i)),
        out_shape=jax.ShapeDtypeStruct((m, n), jnp.float32),
    )(inputs, mem)


# speedup vs baseline: 1.1533x; 1.1533x over previous
import jax
import jax.numpy as jnp
from jax.experimental import pallas as pl

_TN = 4096


def _logits_kernel(x_ref, m_ref, o_ref):
    o_ref[...] = jax.lax.dot_general(
        x_ref[...],
        m_ref[...],
        dimension_numbers=(((1,), (1,)), ((), ())),
        preferred_element_type=jnp.float32,
    )


def kernel(inputs, targets, mem, epoch):
    del targets, epoch
    m, k = inputs.shape
    n = mem.shape[0]
    return pl.pallas_call(
        _logits_kernel,
        grid=(pl.cdiv(n, _TN),),
        in_specs=[
            pl.BlockSpec((m, k), lambda i: (0, 0)),
            pl.BlockSpec((_TN, k), lambda i: (i, 0)),
        ],
        out_specs=pl.BlockSpec((m, _TN), lambda i: (0, i)),
        out_shape=jax.ShapeDtypeStruct((m, n), jnp.float32),
    )(inputs, mem)
